# Initial kernel scaffold; baseline (speedup 1.0000x reference)
import jax
import jax.numpy as jnp
from jax.experimental import pallas as pl


def kernel(fused_proto, base_proto, logits):
    raise NotImplementedError("write your pallas kernel here")


# speedup vs baseline: 77.7188x; 77.7188x over previous
import functools

import jax
import jax.numpy as jnp
from jax import lax
from jax.experimental import pallas as pl
from jax.experimental.pallas import tpu as pltpu

_RATIOS = (0.1, 0.25, 0.5)
_NBITS = 31


def _body(f_ref, b_ref, w_ref, o_ref, *, ks):
    f = f_ref[...]
    d = jnp.abs(f - b_ref[...])
    bits = lax.bitcast_convert_type(d, jnp.int32)
    rows = bits.shape[0]
    wt = jnp.zeros_like(f)
    for j, k in enumerate(ks):
        t = jnp.zeros((rows, 1), jnp.int32)
        for i in range(_NBITS):
            cand = t | jnp.int32(1 << (_NBITS - 1 - i))
            cnt = jnp.sum((bits >= cand).astype(jnp.int32), axis=1,
                          keepdims=True)
            t = jnp.where(cnt >= k, cand, t)
        wj = w_ref[0, j]
        wt = wt + jnp.where(bits >= t, wj, jnp.float32(0.0))
    o_ref[...] = f * wt


def kernel(fused_proto, base_proto, logits):
    q, n, d = fused_proto.shape
    r = q * n
    ks = tuple(max(1, int(ratio * d)) for ratio in _RATIOS)
    w = jax.nn.softmax(logits)
    w_pad = jnp.zeros((8, 128), jnp.float32).at[0, : w.shape[0]].set(w)

    br = min(512, r)
    grid = r // br
    f2 = fused_proto.reshape(r, d)
    b2 = base_proto.reshape(r, d)
    out = pl.pallas_call(
        functools.partial(_body, ks=ks),
        grid=(grid,),
        in_specs=[
            pl.BlockSpec((br, d), lambda i: (i, 0)),
            pl.BlockSpec((br, d), lambda i: (i, 0)),
            pl.BlockSpec((8, 128), lambda i: (0, 0)),
        ],
        out_specs=pl.BlockSpec((br, d), lambda i: (i, 0)),
        out_shape=jax.ShapeDtypeStruct((r, d), jnp.float32),
        compiler_params=pltpu.CompilerParams(
            dimension_semantics=("arbitrary",),
        ),
    )(f2, b2, w_pad)
    return out.reshape(q, n, d)
